# baseline (speedup 1.0000x reference)
import functools

import jax
import jax.numpy as jnp
from jax import lax
from jax.experimental import pallas as pl
from jax.experimental.pallas import tpu as pltpu
from jax.experimental.pallas import tpu_sc as plsc

BATCH = 4096
VOCAB = 100000
NNZ = 204800
UNITS = 128
DENSE_UNITS = 128

NC = 2
NS = 16
L = 16
NW = NC * NS
PER_W = NNZ // NW
CH = 128
NCHUNK = PER_W // CH
GROUPS = UNITS // L
ROWS_PER_TILE = BATCH // NS


NBUF = 4


def _sc_body(w_hbm, col_hbm, row_hbm, val_hbm, out_hbm,
             colv, rowv, valv, buf0, buf1, buf2, buf3,
             acc, gs0, gs1, gs2, gs3, ss0, ss1, ss2, ss3):
    bufs = (buf0, buf1, buf2, buf3)
    gsems = (gs0, gs1, gs2, gs3)
    ssems = (ss0, ss1, ss2, ss3)
    cid = lax.axis_index("c")
    sid = lax.axis_index("s")
    wid = cid * NS + sid

    pltpu.sync_copy(col_hbm.at[wid], colv)
    pltpu.sync_copy(row_hbm.at[wid], rowv)
    pltpu.sync_copy(val_hbm.at[wid], valv)

    zero = jnp.zeros((L,), jnp.float32)

    def _zrow(i, c):
        for g in range(GROUPS):
            buf0[i, pl.ds(g * L, L)] = zero
        return c

    lax.fori_loop(0, CH, _zrow, 0)
    for t in range(ROWS_PER_TILE // CH):
        pltpu.sync_copy(buf0, acc.at[pl.ds(sid * ROWS_PER_TILE + t * CH, CH)])
    plsc.subcore_barrier()

    def _gather(j, b):
        pltpu.async_copy(w_hbm.at[colv.at[j]], bufs[b], gsems[b])

    def _wait_gather(j, b):
        pltpu.make_async_copy(w_hbm.at[colv.at[j]], bufs[b], gsems[b]).wait()

    def _scatter(j, b):
        pltpu.async_copy(bufs[b], acc.at[rowv.at[j]], ssems[b], add=True)

    def _wait_scatter(j, b):
        pltpu.make_async_copy(bufs[b], acc.at[rowv.at[j]], ssems[b]).wait()

    def _scale(j, buf):
        @plsc.parallel_loop(0, CH // L, unroll=2)
        def _blk(i16):
            vv = valv[j, pl.ds(i16 * L, L)]
            for r in range(L):
                vb = jnp.full((L,), vv[r], dtype=jnp.float32)
                i = i16 * L + r
                for g in range(GROUPS):
                    sl = pl.ds(g * L, L)
                    buf[i, sl] = buf[i, sl] * vb

    _gather(0, 0)
    _gather(1, 1)

    def _step(k, c):
        for b in range(NBUF):
            j = NBUF * k + b
            _wait_gather(j, b)
            bn = (b + 2) % NBUF

            @pl.when(j >= 2)
            def _():
                _wait_scatter(j - 2, bn)

            _gather(j + 2, bn)
            _scale(j, bufs[b])
            _scatter(j, b)
        return c

    lax.fori_loop(0, (NCHUNK - 2) // NBUF, _step, 0)

    for j, b in ((NCHUNK - 2, 0), (NCHUNK - 1, 1)):
        _wait_gather(j, b)
        _scale(j, bufs[b])
        _scatter(j, b)
    for j, b in ((NCHUNK - 4, 2), (NCHUNK - 3, 3), (NCHUNK - 2, 0), (NCHUNK - 1, 1)):
        _wait_scatter(j, b)

    plsc.subcore_barrier()
    sl = pl.ds(sid * ROWS_PER_TILE, ROWS_PER_TILE)
    pltpu.sync_copy(acc.at[sl], out_hbm.at[cid, sl])


@functools.partial(jax.jit, static_argnames=())
def _sc_partials(w, cols, rows, vals):
    mesh = plsc.VectorSubcoreMesh(core_axis_name="c", subcore_axis_name="s")
    k = pl.kernel(
        _sc_body,
        out_type=jax.ShapeDtypeStruct((NC, BATCH, UNITS), jnp.float32),
        mesh=mesh,
        scratch_types=[
            pltpu.VMEM((NCHUNK, CH), jnp.int32),
            pltpu.VMEM((NCHUNK, CH), jnp.int32),
            pltpu.VMEM((NCHUNK, CH), jnp.float32),
            pltpu.VMEM((CH, UNITS), jnp.float32),
            pltpu.VMEM((CH, UNITS), jnp.float32),
            pltpu.VMEM((CH, UNITS), jnp.float32),
            pltpu.VMEM((CH, UNITS), jnp.float32),
            pltpu.VMEM_SHARED((BATCH, UNITS), jnp.float32),
        ] + [pltpu.SemaphoreType.DMA] * (2 * NBUF),
    )
    return k(w, cols, rows, vals)


def _tc_body(p_ref, d_ref, b_ref, o_ref):
    o_ref[:, :UNITS] = p_ref[0] + p_ref[1] + b_ref[0:1, :]
    o_ref[:, UNITS:] = d_ref[:]


def _combine(partials, dense_feat, b2d):
    blk = 2048
    grid = BATCH // blk
    return pl.pallas_call(
        _tc_body,
        out_shape=jax.ShapeDtypeStruct((BATCH, UNITS + DENSE_UNITS), jnp.float32),
        grid=(grid,),
        in_specs=[
            pl.BlockSpec((NC, blk, UNITS), lambda i: (0, i, 0)),
            pl.BlockSpec((blk, DENSE_UNITS), lambda i: (i, 0)),
            pl.BlockSpec((1, UNITS), lambda i: (0, 0)),
        ],
        out_specs=pl.BlockSpec((blk, UNITS + DENSE_UNITS), lambda i: (i, 0)),
    )(partials, dense_feat, b2d)


def kernel(sparse_values, sparse_row_ids, sparse_col_ids, dense_feat, W, b):
    cols = sparse_col_ids.astype(jnp.int32).reshape(NW, NCHUNK, CH)
    rows = sparse_row_ids.astype(jnp.int32).reshape(NW, NCHUNK, CH)
    vals = sparse_values.reshape(NW, NCHUNK, CH)
    partials = _sc_partials(W, cols, rows, vals)
    return _combine(partials, dense_feat, b.reshape(1, UNITS))


# speedup vs baseline: 1.0256x; 1.0256x over previous
import functools

import jax
import jax.numpy as jnp
from jax import lax
from jax.experimental import pallas as pl
from jax.experimental.pallas import tpu as pltpu
from jax.experimental.pallas import tpu_sc as plsc

BATCH = 4096
VOCAB = 100000
NNZ = 204800
UNITS = 128
DENSE_UNITS = 128

NC = 2
NS = 16
L = 16
NW = NC * NS
PER_W = NNZ // NW
CH = 128
NCHUNK = PER_W // CH
GROUPS = UNITS // L
ROWS_PER_TILE = BATCH // NS


NBUF = 4


def _sc_body(w_hbm, col_hbm, row_hbm, val_hbm, out_hbm,
             colv, rowv, valv, buf0, buf1, buf2, buf3,
             acc, gs0, gs1, gs2, gs3, ss0, ss1, ss2, ss3):
    bufs = (buf0, buf1, buf2, buf3)
    gsems = (gs0, gs1, gs2, gs3)
    ssems = (ss0, ss1, ss2, ss3)
    cid = lax.axis_index("c")
    sid = lax.axis_index("s")
    wid = cid * NS + sid

    pltpu.sync_copy(col_hbm.at[wid], colv)
    pltpu.sync_copy(row_hbm.at[wid], rowv)
    pltpu.sync_copy(val_hbm.at[wid], valv)

    zero = jnp.zeros((L,), jnp.float32)

    def _zrow(i, c):
        for g in range(GROUPS):
            buf0[i, pl.ds(g * L, L)] = zero
        return c

    lax.fori_loop(0, CH, _zrow, 0)
    for t in range(ROWS_PER_TILE // CH):
        pltpu.sync_copy(buf0, acc.at[pl.ds(sid * ROWS_PER_TILE + t * CH, CH)])
    plsc.subcore_barrier()

    def _gather(j, b):
        pltpu.async_copy(w_hbm.at[colv.at[j]], bufs[b], gsems[b])

    def _wait_gather(j, b):
        pltpu.make_async_copy(w_hbm.at[colv.at[j]], bufs[b], gsems[b]).wait()

    def _scatter(j, b):
        pltpu.async_copy(bufs[b], acc.at[rowv.at[j]], ssems[b], add=True)

    def _wait_scatter(j, b):
        pltpu.make_async_copy(bufs[b], acc.at[rowv.at[j]], ssems[b]).wait()

    def _scale(j, buf):
        @plsc.parallel_loop(0, CH // L, unroll=2)
        def _blk(i16):
            vv = valv[j, pl.ds(i16 * L, L)]
            for r in range(L):
                vb = jnp.full((L,), vv[r], dtype=jnp.float32)
                i = i16 * L + r
                for g in range(GROUPS):
                    sl = pl.ds(g * L, L)
                    buf[i, sl] = buf[i, sl] * vb

    _gather(0, 0)
    _gather(1, 1)

    def _step(k, c):
        for b in range(NBUF):
            j = NBUF * k + b
            _wait_gather(j, b)
            _scale(j, bufs[b])
            _scatter(j, b)
            bn = (b + 2) % NBUF

            @pl.when(j >= 2)
            def _():
                _wait_scatter(j - 2, bn)

            _gather(j + 2, bn)
        return c

    lax.fori_loop(0, (NCHUNK - 2) // NBUF, _step, 0)

    for j, b in ((NCHUNK - 2, 0), (NCHUNK - 1, 1)):
        _wait_gather(j, b)
        _scale(j, bufs[b])
        _scatter(j, b)
    for j, b in ((NCHUNK - 4, 2), (NCHUNK - 3, 3), (NCHUNK - 2, 0), (NCHUNK - 1, 1)):
        _wait_scatter(j, b)

    plsc.subcore_barrier()
    sl = pl.ds(sid * ROWS_PER_TILE, ROWS_PER_TILE)
    pltpu.sync_copy(acc.at[sl], out_hbm.at[cid, sl])


@functools.partial(jax.jit, static_argnames=())
def _sc_partials(w, cols, rows, vals):
    mesh = plsc.VectorSubcoreMesh(core_axis_name="c", subcore_axis_name="s")
    k = pl.kernel(
        _sc_body,
        out_type=jax.ShapeDtypeStruct((NC, BATCH, UNITS), jnp.float32),
        mesh=mesh,
        scratch_types=[
            pltpu.VMEM((NCHUNK, CH), jnp.int32),
            pltpu.VMEM((NCHUNK, CH), jnp.int32),
            pltpu.VMEM((NCHUNK, CH), jnp.float32),
            pltpu.VMEM((CH, UNITS), jnp.float32),
            pltpu.VMEM((CH, UNITS), jnp.float32),
            pltpu.VMEM((CH, UNITS), jnp.float32),
            pltpu.VMEM((CH, UNITS), jnp.float32),
            pltpu.VMEM_SHARED((BATCH, UNITS), jnp.float32),
        ] + [pltpu.SemaphoreType.DMA] * (2 * NBUF),
    )
    return k(w, cols, rows, vals)


def _tc_body(p_ref, d_ref, b_ref, o_ref):
    o_ref[:, :UNITS] = p_ref[0] + p_ref[1] + b_ref[0:1, :]
    o_ref[:, UNITS:] = d_ref[:]


def _combine(partials, dense_feat, b2d):
    blk = 2048
    grid = BATCH // blk
    return pl.pallas_call(
        _tc_body,
        out_shape=jax.ShapeDtypeStruct((BATCH, UNITS + DENSE_UNITS), jnp.float32),
        grid=(grid,),
        in_specs=[
            pl.BlockSpec((NC, blk, UNITS), lambda i: (0, i, 0)),
            pl.BlockSpec((blk, DENSE_UNITS), lambda i: (i, 0)),
            pl.BlockSpec((1, UNITS), lambda i: (0, 0)),
        ],
        out_specs=pl.BlockSpec((blk, UNITS + DENSE_UNITS), lambda i: (i, 0)),
    )(partials, dense_feat, b2d)


def kernel(sparse_values, sparse_row_ids, sparse_col_ids, dense_feat, W, b):
    cols = sparse_col_ids.astype(jnp.int32).reshape(NW, NCHUNK, CH)
    rows = sparse_row_ids.astype(jnp.int32).reshape(NW, NCHUNK, CH)
    vals = sparse_values.reshape(NW, NCHUNK, CH)
    partials = _sc_partials(W, cols, rows, vals)
    return _combine(partials, dense_feat, b.reshape(1, UNITS))
